# baseline (speedup 1.0000x reference)
import jax
import jax.numpy as jnp
from jax.experimental import pallas as pl

N_CAND = 4


def _tree_reduce(x, op):
    half = x.shape[0] // 2
    while half >= 1:
        x = op(x[:half], x[half:])
        half //= 2
    return x


def _lex_tree_min(cv, ci):
    cmin = _tree_reduce(cv, jnp.minimum)
    ik = jnp.where(cv == cmin, ci, ci.shape[0])
    return cmin, _tree_reduce(ik, jnp.minimum)


def _vq_kernel(x_ref, wt2_ref, wsq_ref, wt_ref, q_ref, idx_ref):
    d = x_ref.shape[1]
    n_rows = x_ref.shape[2]
    k_total = wt2_ref.shape[0]

    xb = x_ref[0]

    v = jax.lax.dot_general(
        wt2_ref[...], xb,
        dimension_numbers=(((1,), (0,)), ((), ())),
        preferred_element_type=jnp.float32,
        precision=jax.lax.Precision.HIGHEST) + wsq_ref[...]

    kidx = jax.lax.broadcasted_iota(jnp.int32, (k_total, n_rows), 0)
    cand = []
    vm = v
    for c in range(N_CAND):
        _, ci = _lex_tree_min(vm, kidx)
        cand.append(ci)
        if c + 1 < N_CAND:
            vm = jnp.where(kidx == ci, jnp.inf, vm)

    wt_f32 = wt_ref[...]
    whi = wt_f32.astype(jnp.bfloat16)
    rem = wt_f32 - whi.astype(jnp.float32)
    wmid = rem.astype(jnp.bfloat16)
    wlo = (rem - wmid.astype(jnp.float32)).astype(jnp.bfloat16)
    wsel = []
    for c in range(N_CAND):
        oh = (kidx == cand[c]).astype(jnp.bfloat16)
        sel = None
        for wp in (whi, wmid, wlo):
            p = jax.lax.dot_general(
                wp, oh, dimension_numbers=(((0,), (0,)), ((), ())),
                preferred_element_type=jnp.float32)
            sel = p if sel is None else sel + p
        wsel.append(sel)

    accs = [jnp.zeros((1, n_rows), jnp.float32) for _ in range(N_CAND)]
    for j in range(d):
        xrow = xb[j:j + 1, :]
        for c in range(N_CAND):
            t = xrow - wsel[c][j:j + 1, :]
            accs[c] = accs[c] + t * t

    bv, bi, bq = accs[0], cand[0], wsel[0]
    for c in range(1, N_CAND):
        better = (accs[c] < bv) | ((accs[c] == bv) & (cand[c] < bi))
        bv = jnp.where(better, accs[c], bv)
        bi = jnp.where(better, cand[c], bi)
        bq = jnp.where(better, wsel[c], bq)

    idx_ref[0] = bi
    q_ref[0] = bq


@jax.jit
def kernel(x, weight):
    b, d, h, w = x.shape
    k = weight.shape[1]
    rows = h * w
    xr = x.reshape(b, d, rows)

    wt2 = jnp.transpose(-2.0 * weight, (1, 0))
    wsq = jnp.sum(weight * weight, axis=0)[:, None]
    wt = jnp.transpose(weight, (1, 0))

    q, idx = pl.pallas_call(
        _vq_kernel,
        grid=(b,),
        in_specs=[
            pl.BlockSpec((1, d, rows), lambda i: (i, 0, 0)),
            pl.BlockSpec((k, d), lambda i: (0, 0)),
            pl.BlockSpec((k, 1), lambda i: (0, 0)),
            pl.BlockSpec((k, d), lambda i: (0, 0)),
        ],
        out_specs=[
            pl.BlockSpec((1, d, rows), lambda i: (i, 0, 0)),
            pl.BlockSpec((1, 1, rows), lambda i: (i, 0, 0)),
        ],
        out_shape=[
            jax.ShapeDtypeStruct((b, d, rows), jnp.float32),
            jax.ShapeDtypeStruct((b, 1, rows), jnp.int32),
        ],
    )(xr, wt2, wsq, wt)

    return q.reshape(b, d, h, w), idx.reshape(b, h, w)


# speedup vs baseline: 1.0574x; 1.0574x over previous
import jax
import jax.numpy as jnp
from jax.experimental import pallas as pl

N_CAND = 4


def _lex_tree_min(cv, ci):
    half = cv.shape[0] // 2
    take = cv[:half] <= cv[half:]
    cv = jnp.where(take, cv[:half], cv[half:])
    ci = jnp.where(take, ci[:half], ci[half:])
    half //= 2
    while half >= 1:
        v1, v2 = cv[:half], cv[half:]
        i1, i2 = ci[:half], ci[half:]
        take = (v1 < v2) | ((v1 == v2) & (i1 < i2))
        cv = jnp.where(take, v1, v2)
        ci = jnp.where(take, i1, i2)
        half //= 2
    return cv, ci


def _vq_kernel(x_ref, wt2_ref, wsq_ref, wt_ref, q_ref, idx_ref):
    d = x_ref.shape[1]
    n_rows = x_ref.shape[2]
    k_total = wt2_ref.shape[0]

    xb = x_ref[0]

    v = jax.lax.dot_general(
        wt2_ref[...], xb,
        dimension_numbers=(((1,), (0,)), ((), ())),
        preferred_element_type=jnp.float32,
        precision=jax.lax.Precision.HIGHEST) + wsq_ref[...]

    kidx = jax.lax.broadcasted_iota(jnp.int32, (k_total, n_rows), 0)
    cand = []
    vm = v
    for c in range(N_CAND):
        _, ci = _lex_tree_min(vm, kidx)
        cand.append(ci)
        if c + 1 < N_CAND:
            vm = jnp.where(kidx == ci, jnp.inf, vm)

    wt_f32 = wt_ref[...]
    whi = wt_f32.astype(jnp.bfloat16)
    rem = wt_f32 - whi.astype(jnp.float32)
    wmid = rem.astype(jnp.bfloat16)
    wlo = (rem - wmid.astype(jnp.float32)).astype(jnp.bfloat16)
    wsel = []
    for c in range(N_CAND):
        oh = (kidx == cand[c]).astype(jnp.bfloat16)
        sel = None
        for wp in (whi, wmid, wlo):
            p = jax.lax.dot_general(
                wp, oh, dimension_numbers=(((0,), (0,)), ((), ())),
                preferred_element_type=jnp.float32)
            sel = p if sel is None else sel + p
        wsel.append(sel)

    accs = [jnp.zeros((1, n_rows), jnp.float32) for _ in range(N_CAND)]
    for j in range(d):
        xrow = xb[j:j + 1, :]
        for c in range(N_CAND):
            t = xrow - wsel[c][j:j + 1, :]
            accs[c] = accs[c] + t * t

    bv, bi, bq = accs[0], cand[0], wsel[0]
    for c in range(1, N_CAND):
        better = (accs[c] < bv) | ((accs[c] == bv) & (cand[c] < bi))
        bv = jnp.where(better, accs[c], bv)
        bi = jnp.where(better, cand[c], bi)
        bq = jnp.where(better, wsel[c], bq)

    idx_ref[0] = bi
    q_ref[0] = bq


@jax.jit
def kernel(x, weight):
    b, d, h, w = x.shape
    k = weight.shape[1]
    rows = h * w
    xr = x.reshape(b, d, rows)

    wt2 = jnp.transpose(-2.0 * weight, (1, 0))
    wsq = jnp.sum(weight * weight, axis=0)[:, None]
    wt = jnp.transpose(weight, (1, 0))

    q, idx = pl.pallas_call(
        _vq_kernel,
        grid=(b,),
        in_specs=[
            pl.BlockSpec((1, d, rows), lambda i: (i, 0, 0)),
            pl.BlockSpec((k, d), lambda i: (0, 0)),
            pl.BlockSpec((k, 1), lambda i: (0, 0)),
            pl.BlockSpec((k, d), lambda i: (0, 0)),
        ],
        out_specs=[
            pl.BlockSpec((1, d, rows), lambda i: (i, 0, 0)),
            pl.BlockSpec((1, 1, rows), lambda i: (i, 0, 0)),
        ],
        out_shape=[
            jax.ShapeDtypeStruct((b, d, rows), jnp.float32),
            jax.ShapeDtypeStruct((b, 1, rows), jnp.int32),
        ],
    )(xr, wt2, wsq, wt)

    return q.reshape(b, d, h, w), idx.reshape(b, h, w)
